# baseline (speedup 1.0000x reference)
import functools

import jax
import jax.numpy as jnp
from jax import lax
from jax.experimental import pallas as pl
from jax.experimental.pallas import tpu as pltpu
from jax.experimental.pallas import tpu_sc as plsc

B, R, NUM_NUMERICAL = 4096, 20, 16
N = B * R
NC, NS = 2, 16
NW = NC * NS
ROWS_PER_W = N // NW
CHUNK = 128
NCH = ROWS_PER_W // CHUNK
NBUF = 4

NTAB_A = 5
DIMS_A = (16, 32, 16, 16, 16)
COL0_A = (0, 16, 48, 64, 80)

BBC = 512


def _gather_tables(tabs, idx_v, bufs16, bufs32, sgs, sos, out, base,
                   dims, col0, ncols):
    for t in range(len(tabs)):
        d = dims[t]
        c0 = col0[t]
        tab = tabs[t]
        bufs = bufs32 if d == 32 else bufs16

        def iter_body(i, _, tab=tab, d=d, c0=c0, t=t, bufs=bufs):
            for k in range(NBUF):
                j = i * NBUF + k

                @pl.when(j >= NBUF)
                def _(k=k, d=d, c0=c0, bufs=bufs):
                    pltpu.make_async_copy(
                        bufs[k],
                        out.at[pl.ds(base, CHUNK), pl.ds(c0, d)],
                        sos[k]).wait()

                pltpu.async_copy(tab.at[idx_v.at[t, j]], bufs[k], sgs[k])
            for k in range(NBUF):
                j = i * NBUF + k
                pltpu.make_async_copy(tab.at[idx_v.at[t, 0]], bufs[k],
                                      sgs[k]).wait()
                pltpu.async_copy(
                    bufs[k],
                    out.at[pl.ds(base + j * CHUNK, CHUNK), pl.ds(c0, d)],
                    sos[k])
            return 0

        lax.fori_loop(0, NCH // NBUF, iter_body, 0)
        for k in range(NBUF):
            pltpu.make_async_copy(
                bufs[k],
                out.at[pl.ds(base, CHUNK), pl.ds(c0, d)],
                sos[k]).wait()


def _sc_body_a(idx_hbm, tab0, tab1, tab2, tab3, tab4, out,
               idx_v, b16_0, b16_1, b16_2, b16_3, b32_0, b32_1, b32_2, b32_3,
               sg0, sg1, sg2, sg3, so0, so1, so2, so3):
    wid = lax.axis_index("s") * NC + lax.axis_index("c")
    base = wid * ROWS_PER_W
    pltpu.sync_copy(idx_hbm.at[wid], idx_v)
    _gather_tables((tab0, tab1, tab2, tab3, tab4), idx_v,
                   (b16_0, b16_1, b16_2, b16_3),
                   (b32_0, b32_1, b32_2, b32_3),
                   (sg0, sg1, sg2, sg3), (so0, so1, so2, so3),
                   out, base, DIMS_A, COL0_A, 96)


@functools.partial(
    pl.kernel,
    out_type=jax.ShapeDtypeStruct((N, 96), jnp.float32),
    mesh=plsc.VectorSubcoreMesh(core_axis_name="c", subcore_axis_name="s",
                                num_cores=NC, num_subcores=NS),
    compiler_params=pltpu.CompilerParams(use_tc_tiling_on_sc=False),
    scratch_types=[
        pltpu.VMEM((NTAB_A, NCH, CHUNK), jnp.int32),
        pltpu.VMEM((CHUNK, 16), jnp.float32),
        pltpu.VMEM((CHUNK, 16), jnp.float32),
        pltpu.VMEM((CHUNK, 16), jnp.float32),
        pltpu.VMEM((CHUNK, 16), jnp.float32),
        pltpu.VMEM((CHUNK, 32), jnp.float32),
        pltpu.VMEM((CHUNK, 32), jnp.float32),
        pltpu.VMEM((CHUNK, 32), jnp.float32),
        pltpu.VMEM((CHUNK, 32), jnp.float32),
        pltpu.SemaphoreType.DMA,
        pltpu.SemaphoreType.DMA,
        pltpu.SemaphoreType.DMA,
        pltpu.SemaphoreType.DMA,
        pltpu.SemaphoreType.DMA,
        pltpu.SemaphoreType.DMA,
        pltpu.SemaphoreType.DMA,
        pltpu.SemaphoreType.DMA,
    ],
)
def _sc_gather_a(*args):
    _sc_body_a(*args)


def _sc_body_h(idx_hbm, tab, out,
               idx_v, b0, b1, b2, b3,
               sg0, sg1, sg2, sg3, so0, so1, so2, so3):
    wid = lax.axis_index("s") * NC + lax.axis_index("c")
    base = wid * ROWS_PER_W
    pltpu.sync_copy(idx_hbm.at[wid], idx_v)
    _gather_tables((tab,), idx_v, None, (b0, b1, b2, b3),
                   (sg0, sg1, sg2, sg3), (so0, so1, so2, so3),
                   out, base, (32,), (0,), 32)


@functools.partial(
    pl.kernel,
    out_type=jax.ShapeDtypeStruct((N, 32), jnp.float32),
    mesh=plsc.VectorSubcoreMesh(core_axis_name="c", subcore_axis_name="s",
                                num_cores=NC, num_subcores=NS),
    compiler_params=pltpu.CompilerParams(use_tc_tiling_on_sc=False),
    scratch_types=[
        pltpu.VMEM((1, NCH, CHUNK), jnp.int32),
        pltpu.VMEM((CHUNK, 32), jnp.float32),
        pltpu.VMEM((CHUNK, 32), jnp.float32),
        pltpu.VMEM((CHUNK, 32), jnp.float32),
        pltpu.VMEM((CHUNK, 32), jnp.float32),
        pltpu.SemaphoreType.DMA,
        pltpu.SemaphoreType.DMA,
        pltpu.SemaphoreType.DMA,
        pltpu.SemaphoreType.DMA,
        pltpu.SemaphoreType.DMA,
        pltpu.SemaphoreType.DMA,
        pltpu.SemaphoreType.DMA,
        pltpu.SemaphoreType.DMA,
    ],
)
def _sc_gather_h(*args):
    _sc_body_h(*args)


def _mlp_body(emb3, embh3, xn, w1a, w1h, w1n, b1r, w2s, b2r, out):
    embf = emb3[...].reshape(R * BBC, 96)
    embhf = embh3[...].reshape(R * BBC, 32)
    xc = jnp.concatenate(
        [lax.dot_general(xn[r], w1n[...], (((0,), (0,)), ((), ())),
                         preferred_element_type=jnp.float32)
         for r in range(R)], axis=0)
    h = jnp.maximum(
        jnp.dot(embf, w1a[...], preferred_element_type=jnp.float32)
        + jnp.dot(embhf, w1h[...], preferred_element_type=jnp.float32)
        + xc + b1r[...], 0.0)
    acc = jnp.dot(h[0:BBC], w2s[0], preferred_element_type=jnp.float32)
    for r in range(1, R):
        acc = acc + jnp.dot(h[r * BBC:(r + 1) * BBC], w2s[r],
                            preferred_element_type=jnp.float32)
    out[...] = acc[:, :R] + b2r[...]


def _mlp(emb, embh, x_numt, W1, b1, W2, b2):
    nbb = B // BBC
    w2s = (W2.reshape(1, 64, 1)
           * jax.nn.one_hot(jnp.arange(R), 128,
                            dtype=jnp.float32).reshape(R, 1, 128))
    w1a = jnp.concatenate([W1[0:16], W1[48:128]], axis=0)
    out = pl.pallas_call(
        _mlp_body,
        grid=(nbb,),
        in_specs=[
            pl.BlockSpec((R, BBC, 96), lambda bb: (0, bb, 0)),
            pl.BlockSpec((R, BBC, 32), lambda bb: (0, bb, 0)),
            pl.BlockSpec((R, NUM_NUMERICAL, BBC), lambda bb: (0, 0, bb)),
            pl.BlockSpec((96, 64), lambda bb: (0, 0)),
            pl.BlockSpec((32, 64), lambda bb: (0, 0)),
            pl.BlockSpec((NUM_NUMERICAL, 64), lambda bb: (0, 0)),
            pl.BlockSpec((1, 64), lambda bb: (0, 0)),
            pl.BlockSpec((R, 64, 128), lambda bb: (0, 0, 0)),
            pl.BlockSpec((1, 1), lambda bb: (0, 0)),
        ],
        out_specs=pl.BlockSpec((BBC, R), lambda bb: (bb, 0)),
        out_shape=jax.ShapeDtypeStruct((B, R), jnp.float32),
    )(jnp.reshape(emb, (R, B, 96)), jnp.reshape(embh, (R, B, 32)), x_numt,
      w1a, W1[16:48], W1[128:144], b1.reshape(1, 64), w2s,
      b2.reshape(1, 1))
    return out


def kernel(x_cat_going, x_cat_horse_id, x_cat_jockey_id, x_cat_race_class,
           x_cat_track_id, x_cat_trainer_id, x_num,
           table_going, table_horse_id, table_jockey_id, table_race_class,
           table_track_id, table_trainer_id, W1, b1, W2, b2):
    def widx(x):
        return jnp.reshape(jnp.transpose(x), (NW, 1, NCH, CHUNK))

    idx_a = jnp.concatenate([widx(x) for x in (
        x_cat_going, x_cat_jockey_id, x_cat_race_class,
        x_cat_track_id, x_cat_trainer_id)], axis=1)
    emb = _sc_gather_a(idx_a, table_going, table_jockey_id,
                       table_race_class, table_track_id, table_trainer_id)
    embh = _sc_gather_h(widx(x_cat_horse_id), table_horse_id)
    return _mlp(emb, embh, jnp.transpose(x_num, (1, 2, 0)), W1, b1, W2, b2)


# speedup vs baseline: 1.0841x; 1.0841x over previous
import functools

import jax
import jax.numpy as jnp
from jax import lax
from jax.experimental import pallas as pl
from jax.experimental.pallas import tpu as pltpu
from jax.experimental.pallas import tpu_sc as plsc

B, R, NUM_NUMERICAL = 4096, 20, 16
N = B * R
NC, NS = 2, 16
NW = NC * NS
ROWS_PER_W = N // NW
CHUNK = 128
NCH = ROWS_PER_W // CHUNK
NBUF = 4

NTAB = 6
DIMS_LIST = (16, 32, 32, 16, 16, 16)
COL0 = (0, 16, 48, 80, 96, 112)

BBC = 512


def _sc_body(idx_hbm, tab0, tab1, tab2, tab3, tab4, tab5, out,
             idx_v, b16_0, b16_1, b16_2, b16_3, b32_0, b32_1, b32_2, b32_3,
             sg0, sg1, sg2, sg3, so0, so1, so2, so3):
    wid = lax.axis_index("s") * NC + lax.axis_index("c")
    base = wid * ROWS_PER_W
    tabs = (tab0, tab1, tab2, tab3, tab4, tab5)
    bufs16 = (b16_0, b16_1, b16_2, b16_3)
    bufs32 = (b32_0, b32_1, b32_2, b32_3)
    sgs = (sg0, sg1, sg2, sg3)
    sos = (so0, so1, so2, so3)
    pltpu.sync_copy(idx_hbm.at[wid], idx_v)
    for t in range(NTAB):
        d = DIMS_LIST[t]
        c0 = COL0[t]
        tab = tabs[t]
        bufs = bufs32 if d == 32 else bufs16

        def iter_body(i, _, tab=tab, d=d, c0=c0, t=t, bufs=bufs):
            for k in range(NBUF):
                j = i * NBUF + k

                @pl.when(j >= NBUF)
                def _(k=k, d=d, c0=c0, bufs=bufs):
                    pltpu.make_async_copy(
                        bufs[k],
                        out.at[pl.ds(base, CHUNK), pl.ds(c0, d)],
                        sos[k]).wait()

                pltpu.async_copy(tab.at[idx_v.at[t, j]], bufs[k], sgs[k])
            for k in range(NBUF):
                j = i * NBUF + k
                pltpu.make_async_copy(tab.at[idx_v.at[t, 0]], bufs[k],
                                      sgs[k]).wait()
                pltpu.async_copy(
                    bufs[k],
                    out.at[pl.ds(base + j * CHUNK, CHUNK), pl.ds(c0, d)],
                    sos[k])
            return 0

        lax.fori_loop(0, NCH // NBUF, iter_body, 0)
        for k in range(NBUF):
            pltpu.make_async_copy(
                bufs[k],
                out.at[pl.ds(base, CHUNK), pl.ds(c0, d)],
                sos[k]).wait()


@functools.partial(
    pl.kernel,
    out_type=jax.ShapeDtypeStruct((N, 128), jnp.float32),
    mesh=plsc.VectorSubcoreMesh(core_axis_name="c", subcore_axis_name="s",
                                num_cores=NC, num_subcores=NS),
    compiler_params=pltpu.CompilerParams(use_tc_tiling_on_sc=False),
    scratch_types=[
        pltpu.VMEM((NTAB, NCH, CHUNK), jnp.int32),
        pltpu.VMEM((CHUNK, 16), jnp.float32),
        pltpu.VMEM((CHUNK, 16), jnp.float32),
        pltpu.VMEM((CHUNK, 16), jnp.float32),
        pltpu.VMEM((CHUNK, 16), jnp.float32),
        pltpu.VMEM((CHUNK, 32), jnp.float32),
        pltpu.VMEM((CHUNK, 32), jnp.float32),
        pltpu.VMEM((CHUNK, 32), jnp.float32),
        pltpu.VMEM((CHUNK, 32), jnp.float32),
        pltpu.SemaphoreType.DMA,
        pltpu.SemaphoreType.DMA,
        pltpu.SemaphoreType.DMA,
        pltpu.SemaphoreType.DMA,
        pltpu.SemaphoreType.DMA,
        pltpu.SemaphoreType.DMA,
        pltpu.SemaphoreType.DMA,
        pltpu.SemaphoreType.DMA,
    ],
)
def _sc_gather(*args):
    _sc_body(*args)


def _mlp_body(emb3, xn, w1e, w1n, b1r, w2s, b2r, out):
    embf = emb3[...].reshape(R * BBC, 128)
    xc = jnp.concatenate(
        [lax.dot_general(xn[r], w1n[...], (((0,), (0,)), ((), ())),
                         preferred_element_type=jnp.float32)
         for r in range(R)], axis=0)
    h = jnp.maximum(
        jnp.dot(embf, w1e[...], preferred_element_type=jnp.float32)
        + xc + b1r[...], 0.0)
    acc = jnp.dot(h[0:BBC], w2s[0], preferred_element_type=jnp.float32)
    for r in range(1, R):
        acc = acc + jnp.dot(h[r * BBC:(r + 1) * BBC], w2s[r],
                            preferred_element_type=jnp.float32)
    out[...] = acc[:, :R] + b2r[...]


def _mlp(emb, x_numt, W1, b1, W2, b2):
    nbb = B // BBC
    w2s = (W2.reshape(1, 64, 1)
           * jax.nn.one_hot(jnp.arange(R), 128,
                            dtype=jnp.float32).reshape(R, 1, 128))
    out = pl.pallas_call(
        _mlp_body,
        grid=(nbb,),
        in_specs=[
            pl.BlockSpec((R, BBC, 128), lambda bb: (0, bb, 0)),
            pl.BlockSpec((R, NUM_NUMERICAL, BBC), lambda bb: (0, 0, bb)),
            pl.BlockSpec((128, 64), lambda bb: (0, 0)),
            pl.BlockSpec((NUM_NUMERICAL, 64), lambda bb: (0, 0)),
            pl.BlockSpec((1, 64), lambda bb: (0, 0)),
            pl.BlockSpec((R, 64, 128), lambda bb: (0, 0, 0)),
            pl.BlockSpec((1, 1), lambda bb: (0, 0)),
        ],
        out_specs=pl.BlockSpec((BBC, R), lambda bb: (bb, 0)),
        out_shape=jax.ShapeDtypeStruct((B, R), jnp.float32),
    )(jnp.reshape(emb, (R, B, 128)), x_numt, W1[:128], W1[128:],
      b1.reshape(1, 64), w2s, b2.reshape(1, 1))
    return out


def kernel(x_cat_going, x_cat_horse_id, x_cat_jockey_id, x_cat_race_class,
           x_cat_track_id, x_cat_trainer_id, x_num,
           table_going, table_horse_id, table_jockey_id, table_race_class,
           table_track_id, table_trainer_id, W1, b1, W2, b2):
    idx = jnp.stack([jnp.reshape(jnp.transpose(x), (NW, NCH, CHUNK)) for x in (
        x_cat_going, x_cat_horse_id, x_cat_jockey_id, x_cat_race_class,
        x_cat_track_id, x_cat_trainer_id)], axis=1)
    emb = _sc_gather(idx, table_going, table_horse_id, table_jockey_id,
                     table_race_class, table_track_id, table_trainer_id)
    return _mlp(emb, jnp.transpose(x_num, (1, 2, 0)), W1, b1, W2, b2)
